# baseline (speedup 1.0000x reference)
import functools

import jax
import jax.numpy as jnp
from jax import lax
from jax.experimental import pallas as pl
from jax.experimental.pallas import tpu as pltpu
from jax.experimental.pallas import tpu_sc as plsc

L = 16
N = 64
NV = N // L
NC = 2
NS = 16
NW = NC * NS
GPW = 256 // NW


def _rsqrt_nr(d2, iters):
    i = lax.bitcast_convert_type(d2, jnp.int32)
    i = jnp.int32(0x5F3759DF) - lax.shift_right_logical(i, 1)
    y = lax.bitcast_convert_type(i, jnp.float32)
    hd2 = jnp.float32(0.5) * d2
    for _ in range(iters):
        e = hd2 * y
        y = y * (jnp.float32(1.5) - e * y)
    return y


def _bcast(vecs, i, zero):
    k, l = divmod(i, L)
    idx = zero + jnp.int32(l)
    return jnp.take_along_axis(vecs[k], idx, axis=0, mode="promise_in_bounds")


def _sc_body(x_hbm, wg_hbm, wr_hbm, b_hbm, out_hbm, inp_hbm,
             x_v, wg_v, wr_v, b_v, o_v, sem):
    cid = lax.axis_index("c")
    sid = lax.axis_index("s")
    wid = sid * NC + cid
    base = wid * GPW
    b_idx = lax.shift_right_logical(wid, 2)
    t0 = lax.shift_left(lax.bitwise_and(wid, 3), 3)

    lanes = lax.iota(jnp.int32, L)
    zero = lanes - lanes
    one = zero + jnp.int32(1)

    cp_x = pltpu.make_async_copy(x_hbm.at[b_idx], x_v, sem)
    cp_x.start()
    pltpu.sync_copy(wg_hbm, wg_v)
    pltpu.sync_copy(wr_hbm, wr_v)
    pltpu.sync_copy(b_hbm, b_v)
    cp_x.wait()

    toff0 = pl.multiple_of(lax.shift_left(t0, 1), 2 * GPW)
    pltpu.sync_copy(x_v.at[pl.ds(toff0, 2 * GPW)],
                    inp_hbm.at[b_idx, pl.ds(toff0, 2 * GPW)])

    lanes64 = lax.shift_left(lanes, 6)
    lanes2 = lax.shift_left(lanes, 1)

    def group_body(g, carry):
        toff = lax.shift_left(t0 + g, 1)
        x0 = []
        x1 = []
        for k in range(NV):
            i0 = lanes64 + (toff + jnp.int32(k * L * 64))
            x0.append(plsc.load_gather(x_v, [i0]))
            x1.append(plsc.load_gather(x_v, [i0 + one]))

        def wrow(i):
            xi0 = _bcast(x0, i, zero)
            xi1 = _bcast(x1, i, zero)
            ws = []
            for k in range(NV):
                dx0 = x0[k] - xi0
                dx1 = x1[k] - xi1
                d2 = dx0 * dx0 + dx1 * dx1
                ws.append(jnp.minimum(_rsqrt_nr(d2, 1), jnp.float32(1.0)))
            return ws

        deg = wrow(0)
        w0 = list(deg)
        for i in range(1, N):
            wi = wrow(i)
            deg = [a + b for a, b in zip(deg, wi)]

        dis = [_rsqrt_nr(d, 2) for d in deg]
        cw = [w0[k] * dis[k] for k in range(NV)]
        s0p = cw[0] * x0[0] + cw[1] * x0[1] + cw[2] * x0[2] + cw[3] * x0[3]
        s1p = cw[0] * x1[0] + cw[1] * x1[1] + cw[2] * x1[2] + cw[3] * x1[3]
        s0 = s0p
        s1 = s1p
        for sh in (8, 4, 2, 1):
            idx = lax.bitwise_xor(lanes, jnp.int32(sh))
            s0 = s0 + jnp.take_along_axis(s0, idx, axis=0,
                                          mode="promise_in_bounds")
            s1 = s1 + jnp.take_along_axis(s1, idx, axis=0,
                                          mode="promise_in_bounds")
        dis0 = _bcast(dis, 0, zero)
        x00 = _bcast(x0, 0, zero)
        x01 = _bcast(x1, 0, zero)

        for k in range(NV):
            jk = lanes2 + jnp.int32(k * L * 2)
            wg0 = plsc.load_gather(wg_v, [jk])
            wg1 = plsc.load_gather(wg_v, [jk + one])
            wr0 = plsc.load_gather(wr_v, [jk])
            wr1 = plsc.load_gather(wr_v, [jk + one])
            bb = b_v[pl.ds(k * L, L)]
            o = dis0 * (s0 * wg0 + s1 * wg1) + bb + x00 * wr0 + x01 * wr1
            o_v[g, pl.ds(k * L, L)] = jnp.maximum(o, jnp.float32(0.0))
        return carry

    lax.fori_loop(0, GPW, group_body, 0)
    pltpu.sync_copy(o_v, out_hbm.at[pl.ds(base, GPW)])


@jax.jit
def _encode(x, W_gcn, W_res, b_gcn):
    G = 256
    mesh = plsc.VectorSubcoreMesh(core_axis_name="c", subcore_axis_name="s")
    fn = pl.kernel(
        _sc_body,
        out_type=(
            jax.ShapeDtypeStruct((G, N), jnp.float32),
            jax.ShapeDtypeStruct((8, 64), jnp.float32),
        ),
        mesh=mesh,
        scratch_types=[
            pltpu.VMEM((N * 32 * 2,), jnp.float32),
            pltpu.VMEM((N * 2,), jnp.float32),
            pltpu.VMEM((N * 2,), jnp.float32),
            pltpu.VMEM((N,), jnp.float32),
            pltpu.VMEM((GPW, N), jnp.float32),
            pltpu.SemaphoreType.DMA,
        ],
        compiler_params=pltpu.CompilerParams(needs_layout_passes=False),
    )
    return fn(x, W_gcn, W_res, b_gcn)


def kernel(x, W_res, W_gcn, b_gcn):
    B, n, T, F = x.shape
    emb = W_gcn.shape[0]
    enc0, inp = _encode(
        x.astype(jnp.float32).reshape(B, n * T * F),
        W_gcn.astype(jnp.float32).reshape(emb * F),
        W_res.astype(jnp.float32).reshape(emb * F),
        b_gcn.astype(jnp.float32),
    )
    enc_sp = enc0.reshape(B, T, emb)
    inputs = inp.reshape(B, T, F).astype(x.dtype)
    return (inputs, enc_sp)


# speedup vs baseline: 1.0460x; 1.0460x over previous
import functools

import jax
import jax.numpy as jnp
from jax import lax
from jax.experimental import pallas as pl
from jax.experimental.pallas import tpu as pltpu
from jax.experimental.pallas import tpu_sc as plsc

L = 16
N = 64
NV = N // L
NC = 2
NS = 16
NW = NC * NS
GPW = 256 // NW


def _rsqrt_nr(d2, iters):
    i = lax.bitcast_convert_type(d2, jnp.int32)
    i = jnp.int32(0x5F3759DF) - lax.shift_right_logical(i, 1)
    y = lax.bitcast_convert_type(i, jnp.float32)
    hd2 = jnp.float32(0.5) * d2
    for _ in range(iters):
        e = hd2 * y
        y = y * (jnp.float32(1.5) - e * y)
    return y


def _bcast(vecs, i, zero):
    k, l = divmod(i, L)
    idx = zero + jnp.int32(l)
    return jnp.take_along_axis(vecs[k], idx, axis=0, mode="promise_in_bounds")


def _sc_body(x_hbm, wg_hbm, wr_hbm, b_hbm, out_hbm,
             x_v, wg_v, wr_v, b_v, o_v, sem):
    cid = lax.axis_index("c")
    sid = lax.axis_index("s")
    wid = sid * NC + cid
    base = wid * GPW
    b_idx = lax.shift_right_logical(wid, 2)
    t0 = lax.shift_left(lax.bitwise_and(wid, 3), 3)

    lanes = lax.iota(jnp.int32, L)
    zero = lanes - lanes
    one = zero + jnp.int32(1)

    cp_x = pltpu.make_async_copy(x_hbm.at[b_idx], x_v, sem)
    cp_x.start()
    pltpu.sync_copy(wg_hbm, wg_v)
    pltpu.sync_copy(wr_hbm, wr_v)
    pltpu.sync_copy(b_hbm, b_v)
    cp_x.wait()

    lanes64 = lax.shift_left(lanes, 6)
    lanes2 = lax.shift_left(lanes, 1)

    def group_body(g, carry):
        toff = lax.shift_left(t0 + g, 1)
        x0 = []
        x1 = []
        for k in range(NV):
            i0 = lanes64 + (toff + jnp.int32(k * L * 64))
            x0.append(plsc.load_gather(x_v, [i0]))
            x1.append(plsc.load_gather(x_v, [i0 + one]))

        def wrow_from(xi0, xi1):
            ws = []
            for k in range(NV):
                dx0 = x0[k] - xi0
                dx1 = x1[k] - xi1
                d2 = dx0 * dx0 + dx1 * dx1
                ws.append(jnp.minimum(_rsqrt_nr(d2, 1), jnp.float32(1.0)))
            return ws

        w0 = wrow_from(_bcast(x0, 0, zero), _bcast(x1, 0, zero))

        def row_body(i, deg):
            kb = lax.shift_right_logical(i, 4) + zero
            idx = lax.bitwise_and(i, 15) + zero
            xi0 = jnp.take_along_axis(
                jnp.where(kb < 2,
                          jnp.where(kb < 1, x0[0], x0[1]),
                          jnp.where(kb < 3, x0[2], x0[3])),
                idx, axis=0, mode="promise_in_bounds")
            xi1 = jnp.take_along_axis(
                jnp.where(kb < 2,
                          jnp.where(kb < 1, x1[0], x1[1]),
                          jnp.where(kb < 3, x1[2], x1[3])),
                idx, axis=0, mode="promise_in_bounds")
            wi = wrow_from(xi0, xi1)
            return tuple(a + b for a, b in zip(deg, wi))

        fzero = lax.convert_element_type(zero, jnp.float32)
        deg = list(lax.fori_loop(0, N, row_body, (fzero,) * NV))

        dis = [_rsqrt_nr(d, 2) for d in deg]
        cw = [w0[k] * dis[k] for k in range(NV)]
        s0p = cw[0] * x0[0] + cw[1] * x0[1] + cw[2] * x0[2] + cw[3] * x0[3]
        s1p = cw[0] * x1[0] + cw[1] * x1[1] + cw[2] * x1[2] + cw[3] * x1[3]
        s0 = s0p
        s1 = s1p
        for sh in (8, 4, 2, 1):
            idx = lax.bitwise_xor(lanes, jnp.int32(sh))
            s0 = s0 + jnp.take_along_axis(s0, idx, axis=0,
                                          mode="promise_in_bounds")
            s1 = s1 + jnp.take_along_axis(s1, idx, axis=0,
                                          mode="promise_in_bounds")
        dis0 = _bcast(dis, 0, zero)
        x00 = _bcast(x0, 0, zero)
        x01 = _bcast(x1, 0, zero)

        for k in range(NV):
            jk = lanes2 + jnp.int32(k * L * 2)
            wg0 = plsc.load_gather(wg_v, [jk])
            wg1 = plsc.load_gather(wg_v, [jk + one])
            wr0 = plsc.load_gather(wr_v, [jk])
            wr1 = plsc.load_gather(wr_v, [jk + one])
            bb = b_v[pl.ds(k * L, L)]
            o = dis0 * (s0 * wg0 + s1 * wg1) + bb + x00 * wr0 + x01 * wr1
            o_v[g, pl.ds(k * L, L)] = jnp.maximum(o, jnp.float32(0.0))
        return carry

    lax.fori_loop(0, GPW, group_body, 0)
    pltpu.sync_copy(o_v, out_hbm.at[pl.ds(base, GPW)])


@jax.jit
def _encode(x, W_gcn, W_res, b_gcn):
    G = 256
    mesh = plsc.VectorSubcoreMesh(core_axis_name="c", subcore_axis_name="s")
    fn = pl.kernel(
        _sc_body,
        out_type=jax.ShapeDtypeStruct((G, N), jnp.float32),
        mesh=mesh,
        scratch_types=[
            pltpu.VMEM((N * 32 * 2,), jnp.float32),
            pltpu.VMEM((N * 2,), jnp.float32),
            pltpu.VMEM((N * 2,), jnp.float32),
            pltpu.VMEM((N,), jnp.float32),
            pltpu.VMEM((GPW, N), jnp.float32),
            pltpu.SemaphoreType.DMA,
        ],
        compiler_params=pltpu.CompilerParams(needs_layout_passes=False),
    )
    return fn(x, W_gcn, W_res, b_gcn)


def kernel(x, W_res, W_gcn, b_gcn):
    B, n, T, F = x.shape
    emb = W_gcn.shape[0]
    enc0 = _encode(
        x.astype(jnp.float32).reshape(B, n * T * F),
        W_gcn.astype(jnp.float32).reshape(emb * F),
        W_res.astype(jnp.float32).reshape(emb * F),
        b_gcn.astype(jnp.float32),
    )
    enc_sp = enc0.reshape(B, T, emb)
    inputs = x[:, 0, :, :]
    return (inputs, enc_sp)


# speedup vs baseline: 1.0590x; 1.0124x over previous
import functools

import jax
import jax.numpy as jnp
from jax import lax
from jax.experimental import pallas as pl
from jax.experimental.pallas import tpu as pltpu
from jax.experimental.pallas import tpu_sc as plsc

L = 16
N = 64
NV = N // L
NC = 2
NS = 16
NW = NC * NS
GPW = 256 // NW


def _rsqrt_nr(d2, iters):
    i = lax.bitcast_convert_type(d2, jnp.int32)
    i = jnp.int32(0x5F3759DF) - lax.shift_right_logical(i, 1)
    y = lax.bitcast_convert_type(i, jnp.float32)
    hd2 = jnp.float32(0.5) * d2
    for _ in range(iters):
        e = hd2 * y
        y = y * (jnp.float32(1.5) - e * y)
    return y


def _bcast(vecs, i, zero):
    k, l = divmod(i, L)
    idx = zero + jnp.int32(l)
    return jnp.take_along_axis(vecs[k], idx, axis=0, mode="promise_in_bounds")


def _sc_body(x_hbm, wg_hbm, wr_hbm, b_hbm, out_hbm,
             x_v, wg_v, wr_v, b_v, o_v, sem):
    cid = lax.axis_index("c")
    sid = lax.axis_index("s")
    wid = sid * NC + cid
    base = wid * GPW
    b_idx = lax.shift_right_logical(wid, 2)
    t0 = lax.shift_left(lax.bitwise_and(wid, 3), 3)

    lanes = lax.iota(jnp.int32, L)
    zero = lanes - lanes
    one = zero + jnp.int32(1)

    cp_x = pltpu.make_async_copy(x_hbm.at[b_idx], x_v, sem)
    cp_x.start()
    pltpu.sync_copy(wg_hbm, wg_v)
    pltpu.sync_copy(wr_hbm, wr_v)
    pltpu.sync_copy(b_hbm, b_v)
    cp_x.wait()

    lanes64 = lax.shift_left(lanes, 6)
    lanes2 = lax.shift_left(lanes, 1)

    def group_body(g, carry):
        toff = lax.shift_left(t0 + g, 1)
        x0 = []
        x1 = []
        for k in range(NV):
            i0 = lanes64 + (toff + jnp.int32(k * L * 64))
            x0.append(plsc.load_gather(x_v, [i0]))
            x1.append(plsc.load_gather(x_v, [i0 + one]))

        def wrow_from(xi0, xi1):
            ws = []
            for k in range(NV):
                dx0 = x0[k] - xi0
                dx1 = x1[k] - xi1
                d2 = dx0 * dx0 + dx1 * dx1
                ws.append(jnp.minimum(_rsqrt_nr(d2, 1), jnp.float32(1.0)))
            return ws

        w0 = wrow_from(_bcast(x0, 0, zero), _bcast(x1, 0, zero))

        def row_body(i, deg):
            kb = lax.shift_right_logical(i, 4) + zero
            idx = lax.bitwise_and(i, 15) + zero
            xi0 = jnp.take_along_axis(
                jnp.where(kb < 2,
                          jnp.where(kb < 1, x0[0], x0[1]),
                          jnp.where(kb < 3, x0[2], x0[3])),
                idx, axis=0, mode="promise_in_bounds")
            xi1 = jnp.take_along_axis(
                jnp.where(kb < 2,
                          jnp.where(kb < 1, x1[0], x1[1]),
                          jnp.where(kb < 3, x1[2], x1[3])),
                idx, axis=0, mode="promise_in_bounds")
            wi = wrow_from(xi0, xi1)
            return tuple(a + b for a, b in zip(deg, wi))

        fzero = lax.convert_element_type(zero, jnp.float32)
        deg = list(lax.fori_loop(0, N, row_body, (fzero,) * NV, unroll=4))

        dis = [_rsqrt_nr(d, 2) for d in deg]
        cw = [w0[k] * dis[k] for k in range(NV)]
        s0p = cw[0] * x0[0] + cw[1] * x0[1] + cw[2] * x0[2] + cw[3] * x0[3]
        s1p = cw[0] * x1[0] + cw[1] * x1[1] + cw[2] * x1[2] + cw[3] * x1[3]
        s0 = s0p
        s1 = s1p
        for sh in (8, 4, 2, 1):
            idx = lax.bitwise_xor(lanes, jnp.int32(sh))
            s0 = s0 + jnp.take_along_axis(s0, idx, axis=0,
                                          mode="promise_in_bounds")
            s1 = s1 + jnp.take_along_axis(s1, idx, axis=0,
                                          mode="promise_in_bounds")
        dis0 = _bcast(dis, 0, zero)
        x00 = _bcast(x0, 0, zero)
        x01 = _bcast(x1, 0, zero)

        for k in range(NV):
            jk = lanes2 + jnp.int32(k * L * 2)
            wg0 = plsc.load_gather(wg_v, [jk])
            wg1 = plsc.load_gather(wg_v, [jk + one])
            wr0 = plsc.load_gather(wr_v, [jk])
            wr1 = plsc.load_gather(wr_v, [jk + one])
            bb = b_v[pl.ds(k * L, L)]
            o = dis0 * (s0 * wg0 + s1 * wg1) + bb + x00 * wr0 + x01 * wr1
            o_v[g, pl.ds(k * L, L)] = jnp.maximum(o, jnp.float32(0.0))
        return carry

    lax.fori_loop(0, GPW, group_body, 0)
    pltpu.sync_copy(o_v, out_hbm.at[pl.ds(base, GPW)])


@jax.jit
def _encode(x, W_gcn, W_res, b_gcn):
    G = 256
    mesh = plsc.VectorSubcoreMesh(core_axis_name="c", subcore_axis_name="s")
    fn = pl.kernel(
        _sc_body,
        out_type=jax.ShapeDtypeStruct((G, N), jnp.float32),
        mesh=mesh,
        scratch_types=[
            pltpu.VMEM((N * 32 * 2,), jnp.float32),
            pltpu.VMEM((N * 2,), jnp.float32),
            pltpu.VMEM((N * 2,), jnp.float32),
            pltpu.VMEM((N,), jnp.float32),
            pltpu.VMEM((GPW, N), jnp.float32),
            pltpu.SemaphoreType.DMA,
        ],
        compiler_params=pltpu.CompilerParams(needs_layout_passes=False),
    )
    return fn(x, W_gcn, W_res, b_gcn)


def kernel(x, W_res, W_gcn, b_gcn):
    B, n, T, F = x.shape
    emb = W_gcn.shape[0]
    enc0 = _encode(
        x.astype(jnp.float32).reshape(B, n * T * F),
        W_gcn.astype(jnp.float32).reshape(emb * F),
        W_res.astype(jnp.float32).reshape(emb * F),
        b_gcn.astype(jnp.float32),
    )
    enc_sp = enc0.reshape(B, T, emb)
    inputs = x[:, 0, :, :]
    return (inputs, enc_sp)
